# Initial kernel scaffold; baseline (speedup 1.0000x reference)
import jax
import jax.numpy as jnp
from jax.experimental import pallas as pl


def kernel(x, edge_index0, edge_index1, Wl0, Wr0, g0, b0, rm0, rv0, Wl1, Wr1, g1, b1, rm1, rv1, Wlin, blin):
    raise NotImplementedError("write your pallas kernel here")


# speedup vs baseline: 4.1999x; 4.1999x over previous
import functools

import jax
import jax.numpy as jnp
from jax import lax
from jax.experimental import pallas as pl
from jax.experimental.pallas import tpu as pltpu
from jax.experimental.pallas import tpu_sc as plsc

N0, N1, N2 = 10000, 5000, 2000
E0, E1 = 320000, 160000
D, H, C = 128, 128, 64
EPS = 1e-5

NC, NS, L = 2, 16, 16
NW = NC * NS
CHUNK = 128
TBL = 2048
TRASH = TBL - 1


def _pad_edges(row, col, e_pad):
    e = row.shape[0]
    pad = e_pad - e
    row = jnp.concatenate([row.astype(jnp.int32), jnp.zeros((pad,), jnp.int32)])
    col = jnp.concatenate([col.astype(jnp.int32), jnp.full((pad,), TRASH, jnp.int32)])
    return row, col


@functools.partial(jax.jit, static_argnums=(4, 5))
def _sc_segment_sum(table, row, col, zeros_tbl, n_chunks, clamp):
    mesh = plsc.VectorSubcoreMesh(core_axis_name="c", subcore_axis_name="s")
    epw = n_chunks * CHUNK

    @functools.partial(
        pl.kernel,
        out_type=[
            jax.ShapeDtypeStruct((NC * TBL, D), jnp.float32),
            jax.ShapeDtypeStruct((NW, TBL), jnp.float32),
        ],
        mesh=mesh,
        scratch_types=[
            pltpu.VMEM((CHUNK,), jnp.int32),
            pltpu.VMEM((CHUNK,), jnp.int32),
            pltpu.VMEM((CHUNK, D), jnp.float32),
            pltpu.VMEM((TBL,), jnp.float32),
            pltpu.VMEM_SHARED((TBL, D), jnp.float32),
            pltpu.SemaphoreType.DMA,
        ],
        compiler_params=pltpu.CompilerParams(needs_layout_passes=False),
    )
    def sc_kernel(table_hbm, row_hbm, col_hbm, zeros_hbm,
                  agg_hbm, cnt_hbm, row_v, col_v, rows_v, cnt_v, agg_sh, sem):
        cid = lax.axis_index("c")
        sid = lax.axis_index("s")
        wid = sid * NC + cid
        rows_per_tile = TBL // NS

        pltpu.sync_copy(zeros_hbm.at[pl.ds(sid * rows_per_tile, rows_per_tile)],
                        agg_sh.at[pl.ds(sid * rows_per_tile, rows_per_tile)])
        zvec = jnp.zeros((L,), jnp.float32)

        def zero_cnt(i, _):
            cnt_v[pl.ds(i * L, L)] = zvec
            return 0

        lax.fori_loop(0, TBL // L, zero_cnt, 0)
        plsc.subcore_barrier()

        base = wid * epw
        ones = jnp.ones((L,), jnp.float32)

        def chunk_body(j, _):
            off = base + j * CHUNK
            pltpu.sync_copy(row_hbm.at[pl.ds(off, CHUNK)], row_v)
            pltpu.sync_copy(col_hbm.at[pl.ds(off, CHUNK)], col_v)
            for i in range(CHUNK // L):
                c = col_v[pl.ds(i * L, L)]
                if clamp:
                    c = jnp.where(c < N2, c, TRASH)
                    col_v[pl.ds(i * L, L)] = c
                plsc.addupdate_scatter(cnt_v, [c], ones)
            pltpu.async_copy(table_hbm.at[row_v], rows_v, sem).wait()
            pltpu.sync_copy(rows_v, agg_sh.at[col_v], add=True)
            return 0

        lax.fori_loop(0, n_chunks, chunk_body, 0)
        plsc.subcore_barrier()

        pltpu.sync_copy(agg_sh.at[pl.ds(sid * rows_per_tile, rows_per_tile)],
                        agg_hbm.at[pl.ds(cid * TBL + sid * rows_per_tile,
                                         rows_per_tile)])
        pltpu.sync_copy(cnt_v, cnt_hbm.at[wid])

    return sc_kernel(table, row, col, zeros_tbl)


def _dense_layer0(agg_p, cnt_p, x2, wl_t, wr_t, g, b, rm, rv):
    def body(agg_ref, cnt_ref, x2_ref, wl_ref, wr_ref, g_ref, b_ref,
             rm_ref, rv_ref, out_ref):
        agg = agg_ref[0:N2, :] + agg_ref[TBL:TBL + N2, :]
        cnt = jnp.sum(cnt_ref[...], axis=0)[:N2]
        mean = agg / jnp.clip(cnt, 1.0, None)[:, None]
        h = (jnp.dot(mean, wl_ref[...], preferred_element_type=jnp.float32)
             + jnp.dot(x2_ref[...], wr_ref[...], preferred_element_type=jnp.float32))
        h = (h - rm_ref[...]) / jnp.sqrt(rv_ref[...] + EPS) * g_ref[...] + b_ref[...]
        out_ref[...] = jnp.maximum(h, 0.0)

    return pl.pallas_call(
        body,
        out_shape=jax.ShapeDtypeStruct((N2, H), jnp.float32),
    )(agg_p, cnt_p, x2, wl_t, wr_t, g, b, rm, rv)


def _dense_layer1(agg_p, cnt_p, h0, wl_t, wr_t, g, b, rm, rv, wa, wb, blin):
    def body(agg_ref, cnt_ref, h0_ref, wl_ref, wr_ref, g_ref, b_ref,
             rm_ref, rv_ref, wa_ref, wb_ref, blin_ref, out_ref):
        agg = agg_ref[0:N2, :] + agg_ref[TBL:TBL + N2, :]
        cnt = jnp.sum(cnt_ref[...], axis=0)[:N2]
        mean = agg / jnp.clip(cnt, 1.0, None)[:, None]
        h0 = h0_ref[...]
        h2 = (jnp.dot(mean, wl_ref[...], preferred_element_type=jnp.float32)
              + jnp.dot(h0, wr_ref[...], preferred_element_type=jnp.float32))
        h2 = (h2 - rm_ref[...]) / jnp.sqrt(rv_ref[...] + EPS) * g_ref[...] + b_ref[...]
        h2 = jnp.maximum(h2, 0.0)
        z = (jnp.dot(h0, wa_ref[...], preferred_element_type=jnp.float32)
             + jnp.dot(h2, wb_ref[...], preferred_element_type=jnp.float32)
             + blin_ref[...])
        z = z - jnp.max(z, axis=1, keepdims=True)
        out_ref[...] = z - jnp.log(jnp.sum(jnp.exp(z), axis=1, keepdims=True))

    return pl.pallas_call(
        body,
        out_shape=jax.ShapeDtypeStruct((N2, C), jnp.float32),
    )(agg_p, cnt_p, h0, wl_t, wr_t, g, b, rm, rv, wa, wb, blin)


def kernel(x, edge_index0, edge_index1, Wl0, Wr0, g0, b0, rm0, rv0,
           Wl1, Wr1, g1, b1, rm1, rv1, Wlin, blin):
    x = x.astype(jnp.float32)
    zeros_tbl = jnp.zeros((TBL, D), jnp.float32)

    n_chunks0 = -(-E0 // (NW * CHUNK))
    n_chunks1 = -(-E1 // (NW * CHUNK))
    row0, col0 = _pad_edges(edge_index0[0], edge_index0[1], NW * n_chunks0 * CHUNK)
    row1, col1 = _pad_edges(edge_index1[0], edge_index1[1], NW * n_chunks1 * CHUNK)

    agg0, cnt0 = _sc_segment_sum(x, row0, col0, zeros_tbl, n_chunks0, True)
    h0 = _dense_layer0(agg0, cnt0, x[:N2], Wl0.T, Wr0.T,
                       g0[None, :], b0[None, :], rm0[None, :], rv0[None, :])
    agg1, cnt1 = _sc_segment_sum(h0, row1, col1, zeros_tbl, n_chunks1, False)
    out = _dense_layer1(agg1, cnt1, h0, Wl1.T, Wr1.T,
                        g1[None, :], b1[None, :], rm1[None, :], rv1[None, :],
                        Wlin[:, :H].T, Wlin[:, H:].T, blin[None, :])
    return out


# speedup vs baseline: 4.2062x; 1.0015x over previous
import functools

import jax
import jax.numpy as jnp
from jax import lax
from jax.experimental import pallas as pl
from jax.experimental.pallas import tpu as pltpu
from jax.experimental.pallas import tpu_sc as plsc

N0, N1, N2 = 10000, 5000, 2000
E0, E1 = 320000, 160000
D, H, C = 128, 128, 64
EPS = 1e-5

NC, NS, L = 2, 16, 16
NW = NC * NS
CHUNK = 128
TBL = 2048
TRASH = TBL - 1


def _pad_edges(row, col, e_pad):
    e = row.shape[0]
    pad = e_pad - e
    row = jnp.concatenate([row.astype(jnp.int32), jnp.zeros((pad,), jnp.int32)])
    col = jnp.concatenate([col.astype(jnp.int32), jnp.full((pad,), TRASH, jnp.int32)])
    n_chunks = e_pad // (NW * CHUNK)
    return row.reshape(NW * n_chunks, CHUNK), col.reshape(NW * n_chunks, CHUNK)


@functools.partial(jax.jit, static_argnums=(4, 5))
def _sc_segment_sum(table, row2d, col2d, zeros_tbl, n_chunks, clamp):
    mesh = plsc.VectorSubcoreMesh(core_axis_name="c", subcore_axis_name="s")
    assert n_chunks % 2 == 0

    @functools.partial(
        pl.kernel,
        out_type=[
            jax.ShapeDtypeStruct((NC * TBL, D), jnp.float32),
            jax.ShapeDtypeStruct((NW, TBL), jnp.float32),
        ],
        mesh=mesh,
        scratch_types=[
            pltpu.VMEM((n_chunks, CHUNK), jnp.int32),
            pltpu.VMEM((n_chunks, CHUNK), jnp.int32),
            pltpu.VMEM((CHUNK, D), jnp.float32),
            pltpu.VMEM((CHUNK, D), jnp.float32),
            pltpu.VMEM((TBL,), jnp.float32),
            pltpu.VMEM_SHARED((TBL, D), jnp.float32),
            pltpu.SemaphoreType.DMA,
            pltpu.SemaphoreType.DMA,
        ],
        compiler_params=pltpu.CompilerParams(needs_layout_passes=False),
    )
    def sc_kernel(table_hbm, row_hbm, col_hbm, zeros_hbm,
                  agg_hbm, cnt_hbm, row_v, col_v, rows_a, rows_b, cnt_v,
                  agg_sh, sem_a, sem_b):
        cid = lax.axis_index("c")
        sid = lax.axis_index("s")
        wid = sid * NC + cid
        rows_per_tile = TBL // NS

        pltpu.sync_copy(zeros_hbm.at[pl.ds(sid * rows_per_tile, rows_per_tile)],
                        agg_sh.at[pl.ds(sid * rows_per_tile, rows_per_tile)])
        pltpu.sync_copy(row_hbm.at[pl.ds(wid * n_chunks, n_chunks)], row_v)
        pltpu.sync_copy(col_hbm.at[pl.ds(wid * n_chunks, n_chunks)], col_v)
        zvec = jnp.zeros((L,), jnp.float32)

        def zero_cnt(i, _):
            cnt_v[pl.ds(i * L, L)] = zvec
            return 0

        lax.fori_loop(0, TBL // L, zero_cnt, 0)
        plsc.subcore_barrier()

        ones = jnp.ones((L,), jnp.float32)

        def prep(j):
            for i in range(CHUNK // L):
                c = col_v[j, pl.ds(i * L, L)]
                if clamp:
                    c = jnp.where(c < N2, c, TRASH)
                    col_v[j, pl.ds(i * L, L)] = c
                plsc.addupdate_scatter(cnt_v, [c], ones)

        def start(j, buf, sem):
            pltpu.async_copy(table_hbm.at[row_v.at[j]], buf, sem)

        def drain(j, buf, sem):
            pltpu.make_async_copy(table_hbm.at[row_v.at[j]], buf, sem).wait()
            pltpu.sync_copy(buf, agg_sh.at[col_v.at[j]], add=True)

        prep(0)
        start(0, rows_a, sem_a)
        prep(1)

        def pair(g, _):
            j = 2 * g
            start(j + 1, rows_b, sem_b)

            @pl.when(j + 2 < n_chunks)
            def _():
                prep(j + 2)

            drain(j, rows_a, sem_a)

            @pl.when(j + 2 < n_chunks)
            def _():
                start(j + 2, rows_a, sem_a)

            @pl.when(j + 3 < n_chunks)
            def _():
                prep(j + 3)

            drain(j + 1, rows_b, sem_b)
            return 0

        lax.fori_loop(0, n_chunks // 2, pair, 0)
        plsc.subcore_barrier()

        pltpu.sync_copy(agg_sh.at[pl.ds(sid * rows_per_tile, rows_per_tile)],
                        agg_hbm.at[pl.ds(cid * TBL + sid * rows_per_tile,
                                         rows_per_tile)])
        pltpu.sync_copy(cnt_v, cnt_hbm.at[wid])

    return sc_kernel(table, row2d, col2d, zeros_tbl)


def _dense_layer0(agg_p, cnt_p, x2, wl_t, wr_t, g, b, rm, rv):
    def body(agg_ref, cnt_ref, x2_ref, wl_ref, wr_ref, g_ref, b_ref,
             rm_ref, rv_ref, out_ref):
        agg = agg_ref[0:N2, :] + agg_ref[TBL:TBL + N2, :]
        cnt = jnp.sum(cnt_ref[...], axis=0)[:N2]
        mean = agg / jnp.clip(cnt, 1.0, None)[:, None]
        h = (jnp.dot(mean, wl_ref[...], preferred_element_type=jnp.float32)
             + jnp.dot(x2_ref[...], wr_ref[...], preferred_element_type=jnp.float32))
        h = (h - rm_ref[...]) / jnp.sqrt(rv_ref[...] + EPS) * g_ref[...] + b_ref[...]
        out_ref[...] = jnp.maximum(h, 0.0)

    return pl.pallas_call(
        body,
        out_shape=jax.ShapeDtypeStruct((N2, H), jnp.float32),
    )(agg_p, cnt_p, x2, wl_t, wr_t, g, b, rm, rv)


def _dense_layer1(agg_p, cnt_p, h0, wl_t, wr_t, g, b, rm, rv, wa, wb, blin):
    def body(agg_ref, cnt_ref, h0_ref, wl_ref, wr_ref, g_ref, b_ref,
             rm_ref, rv_ref, wa_ref, wb_ref, blin_ref, out_ref):
        agg = agg_ref[0:N2, :] + agg_ref[TBL:TBL + N2, :]
        cnt = jnp.sum(cnt_ref[...], axis=0)[:N2]
        mean = agg / jnp.clip(cnt, 1.0, None)[:, None]
        h0 = h0_ref[...]
        h2 = (jnp.dot(mean, wl_ref[...], preferred_element_type=jnp.float32)
              + jnp.dot(h0, wr_ref[...], preferred_element_type=jnp.float32))
        h2 = (h2 - rm_ref[...]) / jnp.sqrt(rv_ref[...] + EPS) * g_ref[...] + b_ref[...]
        h2 = jnp.maximum(h2, 0.0)
        z = (jnp.dot(h0, wa_ref[...], preferred_element_type=jnp.float32)
             + jnp.dot(h2, wb_ref[...], preferred_element_type=jnp.float32)
             + blin_ref[...])
        z = z - jnp.max(z, axis=1, keepdims=True)
        out_ref[...] = z - jnp.log(jnp.sum(jnp.exp(z), axis=1, keepdims=True))

    return pl.pallas_call(
        body,
        out_shape=jax.ShapeDtypeStruct((N2, C), jnp.float32),
    )(agg_p, cnt_p, h0, wl_t, wr_t, g, b, rm, rv, wa, wb, blin)


def kernel(x, edge_index0, edge_index1, Wl0, Wr0, g0, b0, rm0, rv0,
           Wl1, Wr1, g1, b1, rm1, rv1, Wlin, blin):
    x = x.astype(jnp.float32)
    zeros_tbl = jnp.zeros((TBL, D), jnp.float32)

    def n_chunks_for(e):
        n = -(-e // (NW * CHUNK))
        return n + (n % 2)

    n_chunks0 = n_chunks_for(E0)
    n_chunks1 = n_chunks_for(E1)
    row0, col0 = _pad_edges(edge_index0[0], edge_index0[1], NW * n_chunks0 * CHUNK)
    row1, col1 = _pad_edges(edge_index1[0], edge_index1[1], NW * n_chunks1 * CHUNK)

    agg0, cnt0 = _sc_segment_sum(x, row0, col0, zeros_tbl, n_chunks0, True)
    h0 = _dense_layer0(agg0, cnt0, x[:N2], Wl0.T, Wr0.T,
                       g0[None, :], b0[None, :], rm0[None, :], rv0[None, :])
    agg1, cnt1 = _sc_segment_sum(h0, row1, col1, zeros_tbl, n_chunks1, False)
    out = _dense_layer1(agg1, cnt1, h0, Wl1.T, Wr1.T,
                        g1[None, :], b1[None, :], rm1[None, :], rv1[None, :],
                        Wlin[:, :H].T, Wlin[:, H:].T, blin[None, :])
    return out


# speedup vs baseline: 10.4863x; 2.4930x over previous
import functools

import jax
import jax.numpy as jnp
from jax import lax
from jax.experimental import pallas as pl
from jax.experimental.pallas import tpu as pltpu
from jax.experimental.pallas import tpu_sc as plsc

N0, N1, N2 = 10000, 5000, 2000
E0, E1 = 320000, 160000
D, H, C = 128, 128, 64
EPS = 1e-5

NC, NS, L = 2, 16, 16
NW = NC * NS
CHUNK = 128
TBL = 2048
TRASH = TBL - 1


def _pad_edges(row, col, e_pad):
    e = row.shape[0]
    pad = e_pad - e
    row = jnp.concatenate([row.astype(jnp.int32), jnp.zeros((pad,), jnp.int32)])
    col = jnp.concatenate([col.astype(jnp.int32), jnp.full((pad,), TRASH, jnp.int32)])
    n_chunks = e_pad // (NW * CHUNK)
    return row.reshape(NW * n_chunks, CHUNK), col.reshape(NW * n_chunks, CHUNK)


@functools.partial(jax.jit, static_argnums=(4, 5, 6))
def _sc_segment_sum(table, row2d, col2d, zeros_tbl, n_chunks, clamp, src_rows):
    mesh = plsc.VectorSubcoreMesh(core_axis_name="c", subcore_axis_name="s")
    assert n_chunks % 2 == 0 and src_rows % NS == 0

    @functools.partial(
        pl.kernel,
        out_type=[
            jax.ShapeDtypeStruct((NC * TBL, D), jnp.float32),
            jax.ShapeDtypeStruct((NW, TBL), jnp.float32),
        ],
        mesh=mesh,
        scratch_types=[
            pltpu.VMEM((n_chunks, CHUNK), jnp.int32),
            pltpu.VMEM((n_chunks, CHUNK), jnp.int32),
            pltpu.VMEM((CHUNK, D), jnp.float32),
            pltpu.VMEM((CHUNK, D), jnp.float32),
            pltpu.VMEM((TBL,), jnp.float32),
            pltpu.VMEM_SHARED((TBL, D), jnp.float32),
            pltpu.VMEM_SHARED((src_rows, D), jnp.float32),
            pltpu.SemaphoreType.DMA,
            pltpu.SemaphoreType.DMA,
        ],
        compiler_params=pltpu.CompilerParams(needs_layout_passes=False),
    )
    def sc_kernel(table_hbm, row_hbm, col_hbm, zeros_hbm,
                  agg_hbm, cnt_hbm, row_v, col_v, rows_a, rows_b, cnt_v,
                  agg_sh, table_sh, sem_a, sem_b):
        cid = lax.axis_index("c")
        sid = lax.axis_index("s")
        wid = sid * NC + cid
        rows_per_tile = TBL // NS
        src_per_tile = src_rows // NS

        pltpu.sync_copy(zeros_hbm.at[pl.ds(sid * rows_per_tile, rows_per_tile)],
                        agg_sh.at[pl.ds(sid * rows_per_tile, rows_per_tile)])
        pltpu.sync_copy(table_hbm.at[pl.ds(sid * src_per_tile, src_per_tile)],
                        table_sh.at[pl.ds(sid * src_per_tile, src_per_tile)])
        pltpu.sync_copy(row_hbm.at[pl.ds(wid * n_chunks, n_chunks)], row_v)
        pltpu.sync_copy(col_hbm.at[pl.ds(wid * n_chunks, n_chunks)], col_v)
        zvec = jnp.zeros((L,), jnp.float32)

        def zero_cnt(i, _):
            cnt_v[pl.ds(i * L, L)] = zvec
            return 0

        lax.fori_loop(0, TBL // L, zero_cnt, 0)
        plsc.subcore_barrier()

        ones = jnp.ones((L,), jnp.float32)

        def prep(j):
            for i in range(CHUNK // L):
                c = col_v[j, pl.ds(i * L, L)]
                if clamp:
                    c = jnp.where(c < N2, c, TRASH)
                    col_v[j, pl.ds(i * L, L)] = c
                plsc.addupdate_scatter(cnt_v, [c], ones)

        def start(j, buf, sem):
            pltpu.async_copy(table_sh.at[row_v.at[j]], buf, sem)

        def drain(j, buf, sem):
            pltpu.make_async_copy(table_sh.at[row_v.at[j]], buf, sem).wait()
            pltpu.sync_copy(buf, agg_sh.at[col_v.at[j]], add=True)

        prep(0)
        start(0, rows_a, sem_a)
        prep(1)

        def pair(g, _):
            j = 2 * g
            start(j + 1, rows_b, sem_b)

            @pl.when(j + 2 < n_chunks)
            def _():
                prep(j + 2)

            drain(j, rows_a, sem_a)

            @pl.when(j + 2 < n_chunks)
            def _():
                start(j + 2, rows_a, sem_a)

            @pl.when(j + 3 < n_chunks)
            def _():
                prep(j + 3)

            drain(j + 1, rows_b, sem_b)
            return 0

        lax.fori_loop(0, n_chunks // 2, pair, 0)
        plsc.subcore_barrier()

        pltpu.sync_copy(agg_sh.at[pl.ds(sid * rows_per_tile, rows_per_tile)],
                        agg_hbm.at[pl.ds(cid * TBL + sid * rows_per_tile,
                                         rows_per_tile)])
        pltpu.sync_copy(cnt_v, cnt_hbm.at[wid])

    return sc_kernel(table, row2d, col2d, zeros_tbl)


def _dense_layer0(agg_p, cnt_p, x2, wl_t, wr_t, g, b, rm, rv):
    def body(agg_ref, cnt_ref, x2_ref, wl_ref, wr_ref, g_ref, b_ref,
             rm_ref, rv_ref, out_ref):
        agg = agg_ref[0:N2, :] + agg_ref[TBL:TBL + N2, :]
        cnt = jnp.sum(cnt_ref[...], axis=0)[:N2]
        mean = agg / jnp.clip(cnt, 1.0, None)[:, None]
        h = (jnp.dot(mean, wl_ref[...], preferred_element_type=jnp.float32)
             + jnp.dot(x2_ref[...], wr_ref[...], preferred_element_type=jnp.float32))
        h = (h - rm_ref[...]) / jnp.sqrt(rv_ref[...] + EPS) * g_ref[...] + b_ref[...]
        out_ref[...] = jnp.maximum(h, 0.0)

    return pl.pallas_call(
        body,
        out_shape=jax.ShapeDtypeStruct((N2, H), jnp.float32),
    )(agg_p, cnt_p, x2, wl_t, wr_t, g, b, rm, rv)


def _dense_layer1(agg_p, cnt_p, h0, wl_t, wr_t, g, b, rm, rv, wa, wb, blin):
    def body(agg_ref, cnt_ref, h0_ref, wl_ref, wr_ref, g_ref, b_ref,
             rm_ref, rv_ref, wa_ref, wb_ref, blin_ref, out_ref):
        agg = agg_ref[0:N2, :] + agg_ref[TBL:TBL + N2, :]
        cnt = jnp.sum(cnt_ref[...], axis=0)[:N2]
        mean = agg / jnp.clip(cnt, 1.0, None)[:, None]
        h0 = h0_ref[...]
        h2 = (jnp.dot(mean, wl_ref[...], preferred_element_type=jnp.float32)
              + jnp.dot(h0, wr_ref[...], preferred_element_type=jnp.float32))
        h2 = (h2 - rm_ref[...]) / jnp.sqrt(rv_ref[...] + EPS) * g_ref[...] + b_ref[...]
        h2 = jnp.maximum(h2, 0.0)
        z = (jnp.dot(h0, wa_ref[...], preferred_element_type=jnp.float32)
             + jnp.dot(h2, wb_ref[...], preferred_element_type=jnp.float32)
             + blin_ref[...])
        z = z - jnp.max(z, axis=1, keepdims=True)
        out_ref[...] = z - jnp.log(jnp.sum(jnp.exp(z), axis=1, keepdims=True))

    return pl.pallas_call(
        body,
        out_shape=jax.ShapeDtypeStruct((N2, C), jnp.float32),
    )(agg_p, cnt_p, h0, wl_t, wr_t, g, b, rm, rv, wa, wb, blin)


def kernel(x, edge_index0, edge_index1, Wl0, Wr0, g0, b0, rm0, rv0,
           Wl1, Wr1, g1, b1, rm1, rv1, Wlin, blin):
    x = x.astype(jnp.float32)
    zeros_tbl = jnp.zeros((TBL, D), jnp.float32)

    def n_chunks_for(e):
        n = -(-e // (NW * CHUNK))
        return n + (n % 2)

    n_chunks0 = n_chunks_for(E0)
    n_chunks1 = n_chunks_for(E1)
    row0, col0 = _pad_edges(edge_index0[0], edge_index0[1], NW * n_chunks0 * CHUNK)
    row1, col1 = _pad_edges(edge_index1[0], edge_index1[1], NW * n_chunks1 * CHUNK)

    tbl0 = jnp.pad(x[:N1], ((0, 120), (0, 0)))
    agg0, cnt0 = _sc_segment_sum(tbl0, row0, col0, zeros_tbl, n_chunks0, True,
                                 N1 + 120)
    h0 = _dense_layer0(agg0, cnt0, x[:N2], Wl0.T, Wr0.T,
                       g0[None, :], b0[None, :], rm0[None, :], rv0[None, :])
    tbl1 = jnp.pad(h0, ((0, TBL - N2), (0, 0)))
    agg1, cnt1 = _sc_segment_sum(tbl1, row1, col1, zeros_tbl, n_chunks1, False,
                                 TBL)
    out = _dense_layer1(agg1, cnt1, h0, Wl1.T, Wr1.T,
                        g1[None, :], b1[None, :], rm1[None, :], rv1[None, :],
                        Wlin[:, :H].T, Wlin[:, H:].T, blin[None, :])
    return out


# speedup vs baseline: 16.1512x; 1.5402x over previous
import functools

import jax
import jax.numpy as jnp
from jax import lax
from jax.experimental import pallas as pl
from jax.experimental.pallas import tpu as pltpu
from jax.experimental.pallas import tpu_sc as plsc

N0, N1, N2 = 10000, 5000, 2000
E0, E1 = 320000, 160000
D, H, C = 128, 128, 64
EPS = 1e-5

NC, NS, L = 2, 16, 16
NW = NC * NS
CHUNK = 128
TBL = 2048
TRASH = TBL - 1


def _pad_edges(row, col, e_pad):
    e = row.shape[0]
    pad = e_pad - e
    row = jnp.concatenate([row.astype(jnp.int32), jnp.zeros((pad,), jnp.int32)])
    col = jnp.concatenate([col.astype(jnp.int32), jnp.full((pad,), TRASH, jnp.int32)])
    return row, col


@functools.partial(jax.jit, static_argnums=(4, 5, 6))
def _sc_segment_sum(table, row2d, col2d, zeros_tbl, n_chunks, clamp, src_rows):
    mesh = plsc.VectorSubcoreMesh(core_axis_name="c", subcore_axis_name="s")
    assert n_chunks % 2 == 0 and src_rows % NS == 0

    @functools.partial(
        pl.kernel,
        out_type=[
            jax.ShapeDtypeStruct((NC * TBL, D), jnp.float32),
            jax.ShapeDtypeStruct((NW, TBL), jnp.float32),
        ],
        mesh=mesh,
        scratch_types=[
            pltpu.VMEM((n_chunks * CHUNK + CHUNK,), jnp.int32),
            pltpu.VMEM((n_chunks * CHUNK + CHUNK,), jnp.int32),
            pltpu.VMEM((CHUNK,), jnp.int32),
            pltpu.VMEM((CHUNK,), jnp.int32),
            pltpu.VMEM((CHUNK, D), jnp.float32),
            pltpu.VMEM((CHUNK, D), jnp.float32),
            pltpu.VMEM((TBL,), jnp.float32),
            pltpu.VMEM_SHARED((TBL, D), jnp.float32),
            pltpu.VMEM_SHARED((src_rows, D), jnp.float32),
            pltpu.SemaphoreType.DMA,
            pltpu.SemaphoreType.DMA,
            pltpu.SemaphoreType.DMA,
            pltpu.SemaphoreType.DMA,
            pltpu.SemaphoreType.DMA,
            pltpu.SemaphoreType.DMA,
        ],
        compiler_params=pltpu.CompilerParams(needs_layout_passes=False),
    )
    def sc_kernel(table_hbm, row_hbm, col_hbm, zeros_hbm,
                  agg_hbm, cnt_hbm, row_f, col_f, cols_a, cols_b,
                  rows_a, rows_b, cnt_v, agg_sh, table_sh, sem_a, sem_b,
                  sem_z, sem_t, sem_r, sem_c):
        cid = lax.axis_index("c")
        sid = lax.axis_index("s")
        wid = sid * NC + cid
        rows_per_tile = TBL // NS
        src_per_tile = src_rows // NS
        epw = n_chunks * CHUNK

        zero_src = zeros_hbm.at[pl.ds(sid * rows_per_tile, rows_per_tile)]
        zero_dst = agg_sh.at[pl.ds(sid * rows_per_tile, rows_per_tile)]
        tbl_src = table_hbm.at[pl.ds(sid * src_per_tile, src_per_tile)]
        tbl_dst = table_sh.at[pl.ds(sid * src_per_tile, src_per_tile)]
        row_src = row_hbm.at[pl.ds(wid * epw, epw)]
        row_dst = row_f.at[pl.ds(0, epw)]
        col_src = col_hbm.at[pl.ds(wid * epw, epw)]
        col_dst = col_f.at[pl.ds(0, epw)]
        pltpu.async_copy(row_src, row_dst, sem_r)
        pltpu.async_copy(col_src, col_dst, sem_c)
        pltpu.async_copy(zero_src, zero_dst, sem_z)
        pltpu.async_copy(tbl_src, tbl_dst, sem_t)
        zvec = jnp.zeros((L,), jnp.float32)

        def zero_cnt(i, _):
            cnt_v[pl.ds(i * L, L)] = zvec
            return 0

        lax.fori_loop(0, TBL // L, zero_cnt, 0)
        pltpu.make_async_copy(row_src, row_dst, sem_r).wait()
        pltpu.make_async_copy(col_src, col_dst, sem_c).wait()

        ones = jnp.ones((L,), jnp.float32)

        if clamp:
            def compact(k, w):
                cvec = col_f[pl.ds(k * L, L)]
                rvec = row_f[pl.ds(k * L, L)]
                m = cvec < N2
                plsc.addupdate_scatter(cnt_v, [cvec], ones, mask=m)
                plsc.store_compressed(row_f.at[pl.ds(w, L)], rvec, mask=m)
                plsc.store_compressed(col_f.at[pl.ds(w, L)], cvec, mask=m)
                return w + jnp.sum(m.astype(jnp.int32))

            w = lax.fori_loop(0, epw // L, compact, 0)
            trash_r = jnp.zeros((L,), jnp.int32)
            trash_c = jnp.full((L,), TRASH, jnp.int32)
            for i in range(CHUNK // L):
                row_f[pl.ds(w + i * L, L)] = trash_r
                col_f[pl.ds(w + i * L, L)] = trash_c
            nc = (w + CHUNK - 1) // CHUNK
        else:
            def count(k, _):
                cvec = col_f[pl.ds(k * L, L)]
                plsc.addupdate_scatter(cnt_v, [cvec], ones)
                return 0

            lax.fori_loop(0, epw // L, count, 0)
            nc = jnp.int32(n_chunks)

        pltpu.make_async_copy(zero_src, zero_dst, sem_z).wait()
        pltpu.make_async_copy(tbl_src, tbl_dst, sem_t).wait()
        plsc.subcore_barrier()

        def start(j, rbuf, cbuf, sem):
            pltpu.async_copy(table_sh.at[row_f.at[pl.ds(j * CHUNK, CHUNK)]],
                             rbuf, sem)
            for i in range(CHUNK // L):
                cbuf[pl.ds(i * L, L)] = col_f[pl.ds(j * CHUNK + i * L, L)]

        def drain(j, rbuf, cbuf, sem):
            pltpu.make_async_copy(
                table_sh.at[row_f.at[pl.ds(j * CHUNK, CHUNK)]], rbuf, sem
            ).wait()
            pltpu.sync_copy(rbuf, agg_sh.at[cbuf], add=True)

        @pl.when(nc > 0)
        def _():
            start(0, rows_a, cols_a, sem_a)

        def pair(g, _):
            j = 2 * g

            @pl.when(j + 1 < nc)
            def _():
                start(j + 1, rows_b, cols_b, sem_b)

            drain(j, rows_a, cols_a, sem_a)

            @pl.when(j + 2 < nc)
            def _():
                start(j + 2, rows_a, cols_a, sem_a)

            @pl.when(j + 1 < nc)
            def _():
                drain(j + 1, rows_b, cols_b, sem_b)

            return 0

        lax.fori_loop(0, (nc + 1) // 2, pair, 0)
        plsc.subcore_barrier()

        pltpu.sync_copy(agg_sh.at[pl.ds(sid * rows_per_tile, rows_per_tile)],
                        agg_hbm.at[pl.ds(cid * TBL + sid * rows_per_tile,
                                         rows_per_tile)])
        pltpu.sync_copy(cnt_v, cnt_hbm.at[wid])

    return sc_kernel(table, row2d, col2d, zeros_tbl)


def _dense_layer0(agg_p, cnt_p, x2, wl_t, wr_t, g, b, rm, rv):
    def body(agg_ref, cnt_ref, x2_ref, wl_ref, wr_ref, g_ref, b_ref,
             rm_ref, rv_ref, out_ref):
        agg = agg_ref[0:N2, :] + agg_ref[TBL:TBL + N2, :]
        cnt = jnp.sum(cnt_ref[...], axis=0)[:N2]
        mean = agg / jnp.clip(cnt, 1.0, None)[:, None]
        h = (jnp.dot(mean, wl_ref[...], preferred_element_type=jnp.float32)
             + jnp.dot(x2_ref[...], wr_ref[...], preferred_element_type=jnp.float32))
        h = (h - rm_ref[...]) / jnp.sqrt(rv_ref[...] + EPS) * g_ref[...] + b_ref[...]
        out_ref[...] = jnp.maximum(h, 0.0)

    return pl.pallas_call(
        body,
        out_shape=jax.ShapeDtypeStruct((N2, H), jnp.float32),
    )(agg_p, cnt_p, x2, wl_t, wr_t, g, b, rm, rv)


def _dense_layer1(agg_p, cnt_p, h0, wl_t, wr_t, g, b, rm, rv, wa, wb, blin):
    def body(agg_ref, cnt_ref, h0_ref, wl_ref, wr_ref, g_ref, b_ref,
             rm_ref, rv_ref, wa_ref, wb_ref, blin_ref, out_ref):
        agg = agg_ref[0:N2, :] + agg_ref[TBL:TBL + N2, :]
        cnt = jnp.sum(cnt_ref[...], axis=0)[:N2]
        mean = agg / jnp.clip(cnt, 1.0, None)[:, None]
        h0 = h0_ref[...]
        h2 = (jnp.dot(mean, wl_ref[...], preferred_element_type=jnp.float32)
              + jnp.dot(h0, wr_ref[...], preferred_element_type=jnp.float32))
        h2 = (h2 - rm_ref[...]) / jnp.sqrt(rv_ref[...] + EPS) * g_ref[...] + b_ref[...]
        h2 = jnp.maximum(h2, 0.0)
        z = (jnp.dot(h0, wa_ref[...], preferred_element_type=jnp.float32)
             + jnp.dot(h2, wb_ref[...], preferred_element_type=jnp.float32)
             + blin_ref[...])
        z = z - jnp.max(z, axis=1, keepdims=True)
        out_ref[...] = z - jnp.log(jnp.sum(jnp.exp(z), axis=1, keepdims=True))

    return pl.pallas_call(
        body,
        out_shape=jax.ShapeDtypeStruct((N2, C), jnp.float32),
    )(agg_p, cnt_p, h0, wl_t, wr_t, g, b, rm, rv, wa, wb, blin)


def kernel(x, edge_index0, edge_index1, Wl0, Wr0, g0, b0, rm0, rv0,
           Wl1, Wr1, g1, b1, rm1, rv1, Wlin, blin):
    x = x.astype(jnp.float32)
    zeros_tbl = jnp.zeros((TBL, D), jnp.float32)

    def n_chunks_for(e):
        n = -(-e // (NW * CHUNK))
        return n + (n % 2)

    n_chunks0 = n_chunks_for(E0)
    n_chunks1 = n_chunks_for(E1)
    row0, col0 = _pad_edges(edge_index0[0], edge_index0[1], NW * n_chunks0 * CHUNK)
    row1, col1 = _pad_edges(edge_index1[0], edge_index1[1], NW * n_chunks1 * CHUNK)

    tbl0 = jnp.pad(x[:N1], ((0, 120), (0, 0)))
    agg0, cnt0 = _sc_segment_sum(tbl0, row0, col0, zeros_tbl, n_chunks0, True,
                                 N1 + 120)
    h0 = _dense_layer0(agg0, cnt0, x[:N2], Wl0.T, Wr0.T,
                       g0[None, :], b0[None, :], rm0[None, :], rv0[None, :])
    tbl1 = jnp.pad(h0, ((0, TBL - N2), (0, 0)))
    agg1, cnt1 = _sc_segment_sum(tbl1, row1, col1, zeros_tbl, n_chunks1, False,
                                 TBL)
    out = _dense_layer1(agg1, cnt1, h0, Wl1.T, Wr1.T,
                        g1[None, :], b1[None, :], rm1[None, :], rv1[None, :],
                        Wlin[:, :H].T, Wlin[:, H:].T, blin[None, :])
    return out


# speedup vs baseline: 16.3044x; 1.0095x over previous
import functools

import jax
import jax.numpy as jnp
from jax import lax
from jax.experimental import pallas as pl
from jax.experimental.pallas import tpu as pltpu
from jax.experimental.pallas import tpu_sc as plsc

N0, N1, N2 = 10000, 5000, 2000
E0, E1 = 320000, 160000
D, H, C = 128, 128, 64
EPS = 1e-5

NC, NS, L = 2, 16, 16
NW = NC * NS
CHUNK = 128
TBL = 2048
TRASH = TBL - 1


def _pad_edges(row, col, e_pad):
    e = row.shape[0]
    pad = e_pad - e
    row = jnp.concatenate([row.astype(jnp.int32), jnp.zeros((pad,), jnp.int32)])
    col = jnp.concatenate([col.astype(jnp.int32), jnp.full((pad,), TRASH, jnp.int32)])
    return row, col


@functools.partial(jax.jit, static_argnums=(4, 5, 6))
def _sc_segment_sum(table, row2d, col2d, zeros_tbl, n_chunks, clamp, src_rows):
    mesh = plsc.VectorSubcoreMesh(core_axis_name="c", subcore_axis_name="s")
    assert n_chunks % 2 == 0 and src_rows % NS == 0

    @functools.partial(
        pl.kernel,
        out_type=[
            jax.ShapeDtypeStruct((NC * TBL, D), jnp.float32),
            jax.ShapeDtypeStruct((NW, TBL), jnp.float32),
        ],
        mesh=mesh,
        scratch_types=[
            pltpu.VMEM((n_chunks * CHUNK + CHUNK,), jnp.int32),
            pltpu.VMEM((n_chunks * CHUNK + CHUNK,), jnp.int32),
            pltpu.VMEM((CHUNK,), jnp.int32),
            pltpu.VMEM((CHUNK,), jnp.int32),
            pltpu.VMEM((CHUNK, D), jnp.float32),
            pltpu.VMEM((CHUNK, D), jnp.float32),
            pltpu.VMEM((TBL,), jnp.float32),
            pltpu.VMEM_SHARED((TBL, D), jnp.float32),
            pltpu.VMEM_SHARED((src_rows, D), jnp.float32),
            pltpu.SemaphoreType.DMA,
            pltpu.SemaphoreType.DMA,
            pltpu.SemaphoreType.DMA,
            pltpu.SemaphoreType.DMA,
            pltpu.SemaphoreType.DMA,
            pltpu.SemaphoreType.DMA,
        ],
        compiler_params=pltpu.CompilerParams(needs_layout_passes=False),
    )
    def sc_kernel(table_hbm, row_hbm, col_hbm, zeros_hbm,
                  agg_hbm, cnt_hbm, row_f, col_f, cols_a, cols_b,
                  rows_a, rows_b, cnt_v, agg_sh, table_sh, sem_a, sem_b,
                  sem_z, sem_t, sem_r, sem_c):
        cid = lax.axis_index("c")
        sid = lax.axis_index("s")
        wid = sid * NC + cid
        rows_per_tile = TBL // NS
        src_per_tile = src_rows // NS
        epw = n_chunks * CHUNK

        zero_src = zeros_hbm.at[pl.ds(sid * rows_per_tile, rows_per_tile)]
        zero_dst = agg_sh.at[pl.ds(sid * rows_per_tile, rows_per_tile)]
        tbl_src = table_hbm.at[pl.ds(sid * src_per_tile, src_per_tile)]
        tbl_dst = table_sh.at[pl.ds(sid * src_per_tile, src_per_tile)]
        row_src = row_hbm.at[pl.ds(wid * epw, epw)]
        row_dst = row_f.at[pl.ds(0, epw)]
        col_src = col_hbm.at[pl.ds(wid * epw, epw)]
        col_dst = col_f.at[pl.ds(0, epw)]
        pltpu.async_copy(row_src, row_dst, sem_r)
        pltpu.async_copy(col_src, col_dst, sem_c)
        pltpu.async_copy(zero_src, zero_dst, sem_z)
        pltpu.async_copy(tbl_src, tbl_dst, sem_t)
        zvec = jnp.zeros((L,), jnp.float32)

        def zero_cnt(i, _):
            cnt_v[pl.ds(i * L, L)] = zvec
            return 0

        lax.fori_loop(0, TBL // L, zero_cnt, 0)
        pltpu.make_async_copy(row_src, row_dst, sem_r).wait()
        pltpu.make_async_copy(col_src, col_dst, sem_c).wait()

        ones = jnp.ones((L,), jnp.float32)

        if clamp:
            def compact(k, w):
                cvec = col_f[pl.ds(k * L, L)]
                rvec = row_f[pl.ds(k * L, L)]
                m = cvec < N2
                plsc.addupdate_scatter(cnt_v, [cvec], ones, mask=m)
                plsc.store_compressed(row_f.at[pl.ds(w, L)], rvec, mask=m)
                plsc.store_compressed(col_f.at[pl.ds(w, L)], cvec, mask=m)
                return w + jnp.sum(m.astype(jnp.int32))

            w = lax.fori_loop(0, epw // L, compact, 0)
            trash_r = jnp.zeros((L,), jnp.int32)
            trash_c = jnp.full((L,), TRASH, jnp.int32)
            for i in range(CHUNK // L):
                row_f[pl.ds(w + i * L, L)] = trash_r
                col_f[pl.ds(w + i * L, L)] = trash_c
            nc = (w + CHUNK - 1) // CHUNK
        else:
            def count(k, _):
                cvec = col_f[pl.ds(k * L, L)]
                plsc.addupdate_scatter(cnt_v, [cvec], ones)
                return 0

            lax.fori_loop(0, epw // L, count, 0)
            nc = jnp.int32(n_chunks)

        pltpu.make_async_copy(zero_src, zero_dst, sem_z).wait()
        pltpu.make_async_copy(tbl_src, tbl_dst, sem_t).wait()
        plsc.subcore_barrier()

        def start(j, rbuf, cbuf, sem):
            pltpu.async_copy(table_sh.at[row_f.at[pl.ds(j * CHUNK, CHUNK)]],
                             rbuf, sem)
            for i in range(CHUNK // L):
                cbuf[pl.ds(i * L, L)] = col_f[pl.ds(j * CHUNK + i * L, L)]

        def drain(j, rbuf, cbuf, sem):
            pltpu.make_async_copy(
                table_sh.at[row_f.at[pl.ds(j * CHUNK, CHUNK)]], rbuf, sem
            ).wait()
            pltpu.sync_copy(rbuf, agg_sh.at[cbuf], add=True)

        @pl.when(nc > 0)
        def _():
            start(0, rows_a, cols_a, sem_a)

        def pair(g, _):
            j = 2 * g

            @pl.when(j + 1 < nc)
            def _():
                start(j + 1, rows_b, cols_b, sem_b)

            drain(j, rows_a, cols_a, sem_a)

            @pl.when(j + 2 < nc)
            def _():
                start(j + 2, rows_a, cols_a, sem_a)

            @pl.when(j + 1 < nc)
            def _():
                drain(j + 1, rows_b, cols_b, sem_b)

            return 0

        lax.fori_loop(0, (nc + 1) // 2, pair, 0)
        plsc.subcore_barrier()

        pltpu.sync_copy(agg_sh.at[pl.ds(sid * rows_per_tile, rows_per_tile)],
                        agg_hbm.at[pl.ds(cid * TBL + sid * rows_per_tile,
                                         rows_per_tile)])
        pltpu.sync_copy(cnt_v, cnt_hbm.at[wid])

    return sc_kernel(table, row2d, col2d, zeros_tbl)


def _dense_layer0(agg_p, cnt_p, x2, wl_t, wr_t, g, b, rm, rv):
    def body(agg_ref, cnt_ref, x2_ref, wl_ref, wr_ref, g_ref, b_ref,
             rm_ref, rv_ref, out_ref):
        agg = agg_ref[0:N2, :] + agg_ref[TBL:TBL + N2, :]
        cnt = jnp.sum(cnt_ref[...], axis=0)[:N2]
        mean = agg / jnp.clip(cnt, 1.0, None)[:, None]
        h = (jnp.dot(mean, wl_ref[...], preferred_element_type=jnp.float32)
             + jnp.dot(x2_ref[...], wr_ref[...], preferred_element_type=jnp.float32))
        h = (h - rm_ref[...]) / jnp.sqrt(rv_ref[...] + EPS) * g_ref[...] + b_ref[...]
        out_ref[0:N2, :] = jnp.maximum(h, 0.0)
        out_ref[N2:TBL, :] = jnp.zeros((TBL - N2, H), jnp.float32)

    return pl.pallas_call(
        body,
        out_shape=jax.ShapeDtypeStruct((TBL, H), jnp.float32),
    )(agg_p, cnt_p, x2, wl_t, wr_t, g, b, rm, rv)


def _dense_layer1(agg_p, cnt_p, h0, wl_t, wr_t, g, b, rm, rv, wa, wb, blin):
    def body(agg_ref, cnt_ref, h0_ref, wl_ref, wr_ref, g_ref, b_ref,
             rm_ref, rv_ref, wa_ref, wb_ref, blin_ref, out_ref):
        agg = agg_ref[0:N2, :] + agg_ref[TBL:TBL + N2, :]
        cnt = jnp.sum(cnt_ref[...], axis=0)[:N2]
        mean = agg / jnp.clip(cnt, 1.0, None)[:, None]
        h0 = h0_ref[0:N2, :]
        h2 = (jnp.dot(mean, wl_ref[...], preferred_element_type=jnp.float32)
              + jnp.dot(h0, wr_ref[...], preferred_element_type=jnp.float32))
        h2 = (h2 - rm_ref[...]) / jnp.sqrt(rv_ref[...] + EPS) * g_ref[...] + b_ref[...]
        h2 = jnp.maximum(h2, 0.0)
        z = (jnp.dot(h0, wa_ref[...], preferred_element_type=jnp.float32)
             + jnp.dot(h2, wb_ref[...], preferred_element_type=jnp.float32)
             + blin_ref[...])
        z = z - jnp.max(z, axis=1, keepdims=True)
        out_ref[...] = z - jnp.log(jnp.sum(jnp.exp(z), axis=1, keepdims=True))

    return pl.pallas_call(
        body,
        out_shape=jax.ShapeDtypeStruct((N2, C), jnp.float32),
    )(agg_p, cnt_p, h0, wl_t, wr_t, g, b, rm, rv, wa, wb, blin)


def kernel(x, edge_index0, edge_index1, Wl0, Wr0, g0, b0, rm0, rv0,
           Wl1, Wr1, g1, b1, rm1, rv1, Wlin, blin):
    x = x.astype(jnp.float32)
    zeros_tbl = jnp.zeros((TBL, D), jnp.float32)

    def n_chunks_for(e):
        n = -(-e // (NW * CHUNK))
        return n + (n % 2)

    n_chunks0 = n_chunks_for(E0)
    n_chunks1 = n_chunks_for(E1)
    row0, col0 = _pad_edges(edge_index0[0], edge_index0[1], NW * n_chunks0 * CHUNK)
    row1, col1 = _pad_edges(edge_index1[0], edge_index1[1], NW * n_chunks1 * CHUNK)

    tbl0 = jnp.pad(x[:N1], ((0, 120), (0, 0)))
    agg0, cnt0 = _sc_segment_sum(tbl0, row0, col0, zeros_tbl, n_chunks0, True,
                                 N1 + 120)
    h0p = _dense_layer0(agg0, cnt0, x[:N2], Wl0.T, Wr0.T,
                        g0[None, :], b0[None, :], rm0[None, :], rv0[None, :])
    agg1, cnt1 = _sc_segment_sum(h0p, row1, col1, zeros_tbl, n_chunks1, False,
                                 TBL)
    out = _dense_layer1(agg1, cnt1, h0p, Wl1.T, Wr1.T,
                        g1[None, :], b1[None, :], rm1[None, :], rv1[None, :],
                        Wlin[:, :H].T, Wlin[:, H:].T, blin[None, :])
    return out
